# baseline (speedup 1.0000x reference)
import functools

import jax
import jax.numpy as jnp
from jax import lax
from jax.experimental import pallas as pl
from jax.experimental.pallas import tpu as pltpu
from jax.experimental.pallas import tpu_sc as plsc

_N = 10000
_E = 320000
_REFRACTORY = 0.33
_RELU_CLIP = 1.0
_NT = 16
_EPT = _E // _NT
_L = 16
_G = 4 * _L
_NCH = 4
_GPC = 78
_GRP = _NCH * _GPC
_CH = _GPC * _G
_TAIL = _EPT - _GRP * _G


def _log1p_newton(z):
    y = 1.0 + z
    t = z * 0.6931472
    for _ in range(4):
        t = t - 1.0 + y * jnp.exp(-t)
    return t


def _sc_kernel(state_hbm, w_hbm, src_hbm, dst_hbm, oid_hbm, bias_hbm,
               act_hbm, out_hbm,
               state_v, srch_v, dst_v, wh_v, srct_v, wt_v, row_v, oid_v,
               bias_v, act_v, sum_v, shared, hits,
               sem, semh, sem_oid, sem_ba, semd0, semd1, semd2, semd3):
    wid = lax.axis_index("s")
    eb = wid * _EPT
    bt = _GRP * _G
    semd = [semd0, semd1, semd2, semd3]
    c_dst = [pltpu.async_copy(dst_hbm.at[pl.ds(eb + k * _CH, _CH)],
                              dst_v.at[pl.ds(k * _CH, _CH)], semd[k])
             for k in range(_NCH)]
    c_tail = pltpu.async_copy(dst_hbm.at[pl.ds(eb + bt, _TAIL)],
                              dst_v.at[pl.ds(bt, _TAIL)], semd3)
    c_srct = pltpu.async_copy(src_hbm.at[pl.ds(eb + bt, _TAIL)], srct_v, sem)
    c_wt = pltpu.async_copy(w_hbm.at[pl.ds(eb + bt, _TAIL)], wt_v, sem)
    c_state = pltpu.async_copy(state_hbm, state_v, sem)
    c_oid = pltpu.async_copy(oid_hbm, oid_v, sem_oid)

    @pl.when(wid == 0)
    def _():
        pltpu.async_copy(bias_hbm, bias_v, sem_ba)
        pltpu.async_copy(act_hbm, act_v, sem_ba)

    c_oid.wait()
    lane = lax.iota(jnp.int32, _L)
    o0 = plsc.load_gather(oid_v, [jnp.full((_L,), 0, jnp.int32)])
    o1 = plsc.load_gather(oid_v, [jnp.full((_L,), 1, jnp.int32)])
    o2 = plsc.load_gather(oid_v, [jnp.full((_L,), 2, jnp.int32)])
    o3 = plsc.load_gather(oid_v, [jnp.full((_L,), 3, jnp.int32)])
    lo = jnp.minimum(jnp.minimum(o0, o1), jnp.minimum(o2, o3))
    hi = jnp.maximum(jnp.maximum(o0, o1), jnp.maximum(o2, o3))
    span = lax.bitcast_convert_type(hi - lo, jnp.uint32)
    zero = jnp.zeros((_L,), jnp.float32)

    def scan_body(g, cnt):
        b = g * _G
        h = None
        for u in range(_G // _L):
            d = dst_v[pl.ds(b + u * _L, _L)]
            ud = lax.bitcast_convert_type(d - lo, jnp.uint32)
            hu = ud <= span
            h = hu if h is None else h | hu
        anyhit = jnp.any(h)

        @pl.when(anyhit)
        def _():
            hits[cnt] = g

        return cnt + anyhit.astype(jnp.int32)

    n_hits = jnp.int32(0)
    for k in range(_NCH):
        c_dst[k].wait()
        if k == _NCH - 1:
            c_tail.wait()
        n_hits = lax.fori_loop(k * _GPC, (k + 1) * _GPC, scan_body, n_hits)

    def drain_one():
        pltpu.make_async_copy(src_hbm.at[pl.ds(eb, _G)],
                              srch_v.at[pl.ds(0, _G)], semh).wait()
        pltpu.make_async_copy(w_hbm.at[pl.ds(eb, _G)],
                              wh_v.at[pl.ds(0, _G)], semh).wait()

    def fetch_body(i, carry):
        b = hits[i] * _G
        pltpu.async_copy(src_hbm.at[pl.ds(eb + b, _G)],
                         srch_v.at[pl.ds(i * _G, _G)], semh)
        pltpu.async_copy(w_hbm.at[pl.ds(eb + b, _G)],
                         wh_v.at[pl.ds(i * _G, _G)], semh)
        pl.when(i >= 8)(drain_one)
        return carry

    lax.fori_loop(0, n_hits, fetch_body, jnp.int32(0))

    def drain_body(i, carry):
        drain_one()
        return carry

    lax.fori_loop(0, jnp.minimum(n_hits, 8), drain_body, jnp.int32(0))
    c_state.wait()
    c_srct.wait()
    c_wt.wait()

    def accum(args, d, s, w):
        a0, a1, a2, a3 = args
        m = w * plsc.load_gather(state_v, [s])
        return (a0 + jnp.where(d == o0, m, zero),
                a1 + jnp.where(d == o1, m, zero),
                a2 + jnp.where(d == o2, m, zero),
                a3 + jnp.where(d == o3, m, zero))

    def hit_body(i, carry):
        b = hits[i] * _G
        hb = i * _G
        for u in range(_G // _L):
            carry = accum(carry,
                          dst_v[pl.ds(b + u * _L, _L)],
                          srch_v[pl.ds(hb + u * _L, _L)],
                          wh_v[pl.ds(hb + u * _L, _L)])
        return carry

    acc = lax.fori_loop(0, n_hits, hit_body, (zero, zero, zero, zero))
    acc = accum(acc, dst_v[pl.ds(bt, _L)], srct_v[pl.ds(0, _L)],
                wt_v[pl.ds(0, _L)])
    a0, a1, a2, a3 = accum(acc, dst_v[pl.ds(bt + _L, _L)],
                           srct_v[pl.ds(_L, _L)], wt_v[pl.ds(_L, _L)])

    t0, t1, t2, t3 = jnp.sum(a0), jnp.sum(a1), jnp.sum(a2), jnp.sum(a3)
    row = jnp.where(lane == 0, t0,
          jnp.where(lane == 1, t1,
          jnp.where(lane == 2, t2,
          jnp.where(lane == 3, t3, 0.0)))) * _REFRACTORY
    row_v[...] = row
    pltpu.sync_copy(row_v, shared.at[wid])
    plsc.subcore_barrier()

    @pl.when(wid == 0)
    def _():
        pltpu.make_async_copy(bias_hbm, bias_v, sem_ba).wait()
        pltpu.make_async_copy(act_hbm, act_v, sem_ba).wait()
        pltpu.sync_copy(shared, sum_v)
        x = sum_v[0, :]
        for i in range(1, _NT):
            x = x + sum_v[i, :]
        oid_vec = plsc.load_gather(oid_v, [jnp.minimum(lane, 3)])
        x = x + plsc.load_gather(state_v, [oid_vec]) * _REFRACTORY
        x = x + plsc.load_gather(bias_v, [oid_vec])
        a = plsc.load_gather(act_v, [oid_vec])
        r = x
        r = jnp.where(a == 1, jnp.maximum(x, 0.0), r)
        r = jnp.where(a == 2, jnp.where(x >= 0, x, 0.01 * x), r)
        r = jnp.where(a == 3, jnp.clip(x, 0.0, _RELU_CLIP), r)
        ez = jnp.exp(-2.0 * jnp.abs(x))
        th = (1.0 - ez) / (1.0 + ez)
        r = jnp.where(a == 4, jnp.where(x >= 0, th, -th), r)
        r = jnp.where(a == 5, 1.0 / (1.0 + jnp.exp(-x)), r)
        sp = jnp.maximum(x, 0.0) + _log1p_newton(jnp.exp(-jnp.abs(x)))
        r = jnp.where(a == 6, sp, r)
        r = jnp.where(a == 7, jnp.abs(x), r)
        row_v[...] = r
        pltpu.sync_copy(row_v.at[pl.ds(0, 4)], out_hbm)


_sc_call = functools.partial(
    pl.kernel,
    mesh=plsc.VectorSubcoreMesh(core_axis_name="c", subcore_axis_name="s",
                                num_cores=1),
    compiler_params=pltpu.CompilerParams(needs_layout_passes=False),
    out_type=jax.ShapeDtypeStruct((4,), jnp.float32),
    scratch_types=[
        pltpu.VMEM((_N,), jnp.float32),
        pltpu.VMEM((_GRP * _G,), jnp.int32),
        pltpu.VMEM((_EPT,), jnp.int32),
        pltpu.VMEM((_GRP * _G,), jnp.float32),
        pltpu.VMEM((_TAIL,), jnp.int32),
        pltpu.VMEM((_TAIL,), jnp.float32),
        pltpu.VMEM((_L,), jnp.float32),
        pltpu.VMEM((4,), jnp.int32),
        pltpu.VMEM((_N,), jnp.float32),
        pltpu.VMEM((_N,), jnp.int32),
        pltpu.VMEM((_NT, _L), jnp.float32),
        pltpu.VMEM_SHARED((_NT, _L), jnp.float32),
        pltpu.SMEM((_GRP + 1,), jnp.int32),
        pltpu.SemaphoreType.DMA,
        pltpu.SemaphoreType.DMA,
        pltpu.SemaphoreType.DMA,
        pltpu.SemaphoreType.DMA,
        pltpu.SemaphoreType.DMA,
        pltpu.SemaphoreType.DMA,
        pltpu.SemaphoreType.DMA,
        pltpu.SemaphoreType.DMA,
    ],
)(_sc_kernel)


def kernel(input, state, weights, biases, src, dst, act_ids, output_ids):
    del input
    src = src.astype(jnp.int32)
    dst = dst.astype(jnp.int32)
    return _sc_call(state, weights, src, dst, output_ids.astype(jnp.int32),
                    biases, act_ids)


# speedup vs baseline: 1.0206x; 1.0206x over previous
import functools

import jax
import jax.numpy as jnp
from jax import lax
from jax.experimental import pallas as pl
from jax.experimental.pallas import tpu as pltpu
from jax.experimental.pallas import tpu_sc as plsc

_N = 10000
_E = 320000
_REFRACTORY = 0.33
_RELU_CLIP = 1.0
_NT = 16
_EPT = _E // _NT
_L = 16
_G = 4 * _L
_NCH = 4
_GPC = 78
_GRP = _NCH * _GPC
_CH = _GPC * _G
_TAIL = _EPT - _GRP * _G


def _log1p_newton(z):
    y = 1.0 + z
    t = z * 0.6931472
    for _ in range(4):
        t = t - 1.0 + y * jnp.exp(-t)
    return t


def _sc_kernel(state_hbm, w_hbm, src_hbm, dst_hbm, oid_hbm, bias_hbm,
               act_hbm, out_hbm,
               state_v, src_v, dst_v, w_v, row_v, oid_v, bias_v,
               act_v, sum_v, shared, hits,
               sem, sem_oid, sem_ba, semd0, semd1, semd2, semd3):
    wid = lax.axis_index("s")
    eb = wid * _EPT
    semd = [semd0, semd1, semd2, semd3]
    c_dst = [pltpu.async_copy(dst_hbm.at[pl.ds(eb + k * _CH, _CH)],
                              dst_v.at[pl.ds(k * _CH, _CH)], semd[k])
             for k in range(_NCH)]
    c_tail = pltpu.async_copy(dst_hbm.at[pl.ds(eb + _GRP * _G, _TAIL)],
                              dst_v.at[pl.ds(_GRP * _G, _TAIL)], semd3)
    c_src = pltpu.async_copy(src_hbm.at[pl.ds(eb, _EPT)], src_v, sem)
    c_w = pltpu.async_copy(w_hbm.at[pl.ds(eb, _EPT)], w_v, sem)
    c_state = pltpu.async_copy(state_hbm, state_v, sem)
    c_oid = pltpu.async_copy(oid_hbm, oid_v, sem_oid)

    @pl.when(wid == 0)
    def _():
        pltpu.async_copy(bias_hbm, bias_v, sem_ba)
        pltpu.async_copy(act_hbm, act_v, sem_ba)

    c_oid.wait()
    lane = lax.iota(jnp.int32, _L)
    o0 = plsc.load_gather(oid_v, [jnp.full((_L,), 0, jnp.int32)])
    o1 = plsc.load_gather(oid_v, [jnp.full((_L,), 1, jnp.int32)])
    o2 = plsc.load_gather(oid_v, [jnp.full((_L,), 2, jnp.int32)])
    o3 = plsc.load_gather(oid_v, [jnp.full((_L,), 3, jnp.int32)])
    lo = jnp.minimum(jnp.minimum(o0, o1), jnp.minimum(o2, o3))
    hi = jnp.maximum(jnp.maximum(o0, o1), jnp.maximum(o2, o3))
    span = lax.bitcast_convert_type(hi - lo, jnp.uint32)
    zero = jnp.zeros((_L,), jnp.float32)

    def scan_body(g, cnt):
        b = g * _G
        h = None
        for u in range(_G // _L):
            d = dst_v[pl.ds(b + u * _L, _L)]
            ud = lax.bitcast_convert_type(d - lo, jnp.uint32)
            hu = ud <= span
            h = hu if h is None else h | hu
        anyhit = jnp.any(h)

        @pl.when(anyhit)
        def _():
            hits[cnt] = g

        return cnt + anyhit.astype(jnp.int32)

    n_hits = jnp.int32(0)
    for k in range(_NCH):
        c_dst[k].wait()
        if k == _NCH - 1:
            c_tail.wait()
        n_hits = lax.fori_loop(k * _GPC, (k + 1) * _GPC, scan_body, n_hits)

    c_src.wait()
    c_w.wait()
    c_state.wait()

    def accum(args, b, d):
        a0, a1, a2, a3 = args
        s = src_v[pl.ds(b, _L)]
        w = w_v[pl.ds(b, _L)]
        m = w * plsc.load_gather(state_v, [s])
        return (a0 + jnp.where(d == o0, m, zero),
                a1 + jnp.where(d == o1, m, zero),
                a2 + jnp.where(d == o2, m, zero),
                a3 + jnp.where(d == o3, m, zero))

    def hit_body(i, carry):
        b = hits[i] * _G
        for u in range(_G // _L):
            carry = accum(carry, b + u * _L, dst_v[pl.ds(b + u * _L, _L)])
        return carry

    acc = lax.fori_loop(0, n_hits, hit_body, (zero, zero, zero, zero))
    bt = _GRP * _G
    acc = accum(acc, bt, dst_v[pl.ds(bt, _L)])
    a0, a1, a2, a3 = accum(acc, bt + _L, dst_v[pl.ds(bt + _L, _L)])

    t0, t1, t2, t3 = jnp.sum(a0), jnp.sum(a1), jnp.sum(a2), jnp.sum(a3)
    row = jnp.where(lane == 0, t0,
          jnp.where(lane == 1, t1,
          jnp.where(lane == 2, t2,
          jnp.where(lane == 3, t3, 0.0)))) * _REFRACTORY
    row_v[...] = row
    pltpu.sync_copy(row_v, shared.at[wid])
    plsc.subcore_barrier()

    @pl.when(wid == 0)
    def _():
        pltpu.make_async_copy(bias_hbm, bias_v, sem_ba).wait()
        pltpu.make_async_copy(act_hbm, act_v, sem_ba).wait()
        pltpu.sync_copy(shared, sum_v)
        x = sum_v[0, :]
        for i in range(1, _NT):
            x = x + sum_v[i, :]
        oid_vec = plsc.load_gather(oid_v, [jnp.minimum(lane, 3)])
        x = x + plsc.load_gather(state_v, [oid_vec]) * _REFRACTORY
        x = x + plsc.load_gather(bias_v, [oid_vec])
        a = plsc.load_gather(act_v, [oid_vec])
        r = x
        r = jnp.where(a == 1, jnp.maximum(x, 0.0), r)
        r = jnp.where(a == 2, jnp.where(x >= 0, x, 0.01 * x), r)
        r = jnp.where(a == 3, jnp.clip(x, 0.0, _RELU_CLIP), r)
        ez = jnp.exp(-2.0 * jnp.abs(x))
        th = (1.0 - ez) / (1.0 + ez)
        r = jnp.where(a == 4, jnp.where(x >= 0, th, -th), r)
        r = jnp.where(a == 5, 1.0 / (1.0 + jnp.exp(-x)), r)
        sp = jnp.maximum(x, 0.0) + _log1p_newton(jnp.exp(-jnp.abs(x)))
        r = jnp.where(a == 6, sp, r)
        r = jnp.where(a == 7, jnp.abs(x), r)
        row_v[...] = r
        pltpu.sync_copy(row_v, out_hbm)


_sc_call = functools.partial(
    pl.kernel,
    mesh=plsc.VectorSubcoreMesh(core_axis_name="c", subcore_axis_name="s",
                                num_cores=1),
    compiler_params=pltpu.CompilerParams(needs_layout_passes=False),
    out_type=jax.ShapeDtypeStruct((_L,), jnp.float32),
    scratch_types=[
        pltpu.VMEM((_N,), jnp.float32),
        pltpu.VMEM((_EPT,), jnp.int32),
        pltpu.VMEM((_EPT,), jnp.int32),
        pltpu.VMEM((_EPT,), jnp.float32),
        pltpu.VMEM((_L,), jnp.float32),
        pltpu.VMEM((_L,), jnp.int32),
        pltpu.VMEM((_N,), jnp.float32),
        pltpu.VMEM((_N,), jnp.int32),
        pltpu.VMEM((_NT, _L), jnp.float32),
        pltpu.VMEM_SHARED((_NT, _L), jnp.float32),
        pltpu.SMEM((_GRP + 1,), jnp.int32),
        pltpu.SemaphoreType.DMA,
        pltpu.SemaphoreType.DMA,
        pltpu.SemaphoreType.DMA,
        pltpu.SemaphoreType.DMA,
        pltpu.SemaphoreType.DMA,
        pltpu.SemaphoreType.DMA,
        pltpu.SemaphoreType.DMA,
    ],
)(_sc_kernel)


def kernel(input, state, weights, biases, src, dst, act_ids, output_ids):
    del input
    src = src.astype(jnp.int32)
    dst = dst.astype(jnp.int32)
    oid16 = jnp.concatenate(
        [output_ids.astype(jnp.int32),
         jnp.zeros((_L - output_ids.shape[0],), jnp.int32)])
    res = _sc_call(state, weights, src, dst, oid16, biases, act_ids)
    return res[:4]
